# Initial kernel scaffold; baseline (speedup 1.0000x reference)
import jax
import jax.numpy as jnp
from jax.experimental import pallas as pl


def kernel(x, table):
    raise NotImplementedError("write your pallas kernel here")


# speedup vs baseline: 1.1026x; 1.1026x over previous
import functools

import jax
import jax.numpy as jnp
from jax import lax
from jax.experimental import pallas as pl
from jax.experimental.pallas import tpu as pltpu
from jax.experimental.pallas import tpu_sc as plsc

BATCH = 16384
HIST = 50
EMBED_DIM = 32
B = BATCH * HIST
NC, NS = 2, 16
NW = NC * NS
PER_W = B // NW
CHUNK = 128
K = 8
GROUP = CHUNK * K
N_GROUPS = PER_W // GROUP
IDX_ROWS = PER_W // CHUNK


def _body(x_hbm, table_hbm, out_hbm, idx_v, rows_v, gsem):
    wid = lax.axis_index("s") * NC + lax.axis_index("c")
    base = wid * PER_W

    pltpu.sync_copy(x_hbm.at[pl.ds(wid * IDX_ROWS, IDX_ROWS)], idx_v)

    def group(g, _):
        copies = []
        for b in range(K):
            row = g * K + b
            copies.append(
                pltpu.async_copy(
                    table_hbm.at[idx_v.at[row]],
                    rows_v.at[pl.ds(b * CHUNK, CHUNK)],
                    gsem,
                )
            )
        for cp in copies:
            cp.wait()
        pltpu.sync_copy(rows_v, out_hbm.at[pl.ds(base + g * GROUP, GROUP)])
        return _

    lax.fori_loop(0, N_GROUPS, group, None)


@jax.jit
def _lookup(x2d, table):
    mesh = plsc.VectorSubcoreMesh(core_axis_name="c", subcore_axis_name="s")
    return pl.kernel(
        _body,
        out_type=jax.ShapeDtypeStruct((B, EMBED_DIM), jnp.float32),
        mesh=mesh,
        scratch_types=[
            pltpu.VMEM((IDX_ROWS, CHUNK), jnp.int32),
            pltpu.VMEM((GROUP, EMBED_DIM), jnp.float32),
            pltpu.SemaphoreType.DMA,
        ],
        compiler_params=pltpu.CompilerParams(use_tc_tiling_on_sc=False),
    )(x2d, table)


def kernel(x, table):
    x2d = x.astype(jnp.int32).reshape(B // CHUNK, CHUNK)
    out = _lookup(x2d, table)
    return out.reshape(BATCH, HIST, EMBED_DIM)


# speedup vs baseline: 1.1129x; 1.0093x over previous
import jax
import jax.numpy as jnp
from jax import lax
from jax.experimental import pallas as pl
from jax.experimental.pallas import tpu as pltpu
from jax.experimental.pallas import tpu_sc as plsc

BATCH = 16384
HIST = 50
EMBED_DIM = 32
B = BATCH * HIST
NC, NS = 2, 16
NW = NC * NS
PER_W = B // NW
CHUNK = 128
K = 5
GROUP = CHUNK * K
NBUF = 4
N_GROUPS = PER_W // GROUP
PAIRS = N_GROUPS // NBUF
IDX_ROWS = PER_W // CHUNK
AHEAD = NBUF - 1


def _body(x_hbm, table_hbm, out_hbm, idx_v, rows_v, gsem, ssem):
    wid = lax.axis_index("s") * NC + lax.axis_index("c")
    base = wid * PER_W

    pltpu.sync_copy(x_hbm.at[pl.ds(wid * IDX_ROWS, IDX_ROWS)], idx_v)

    def fire(g, j):
        for b in range(K):
            pltpu.async_copy(
                table_hbm.at[idx_v.at[g * K + b]],
                rows_v.at[j].at[pl.ds(b * CHUNK, CHUNK)],
                gsem.at[j],
            )

    def wait_gathers(j):
        pltpu.make_async_copy(
            table_hbm.at[pl.ds(0, GROUP)], rows_v.at[j], gsem.at[j]
        ).wait()

    def store(g, j):
        pltpu.async_copy(
            rows_v.at[j], out_hbm.at[pl.ds(base + g * GROUP, GROUP)], ssem.at[j]
        )

    def wait_store(j):
        pltpu.make_async_copy(
            rows_v.at[j], out_hbm.at[pl.ds(base, GROUP)], ssem.at[j]
        ).wait()

    for j in range(AHEAD):
        fire(j, j)

    def pair(p, _):
        for j in range(NBUF):
            g = p * NBUF + j
            ja = (j + AHEAD) % NBUF
            ahead = g + AHEAD

            @pl.when(ahead < N_GROUPS)
            def _():
                @pl.when(g >= 1)
                def _():
                    wait_store(ja)

                fire(ahead, ja)

            wait_gathers(j)
            store(g, j)
        return _

    lax.fori_loop(0, PAIRS, pair, None)

    for j in range(NBUF):
        wait_store(j)


@jax.jit
def _lookup(x2d, table):
    mesh = plsc.VectorSubcoreMesh(core_axis_name="c", subcore_axis_name="s")
    return pl.kernel(
        _body,
        out_type=jax.ShapeDtypeStruct((B, EMBED_DIM), jnp.float32),
        mesh=mesh,
        scratch_types=[
            pltpu.VMEM((IDX_ROWS, CHUNK), jnp.int32),
            pltpu.VMEM((NBUF, GROUP, EMBED_DIM), jnp.float32),
            pltpu.SemaphoreType.DMA((NBUF,)),
            pltpu.SemaphoreType.DMA((NBUF,)),
        ],
        compiler_params=pltpu.CompilerParams(use_tc_tiling_on_sc=False),
    )(x2d, table)


def kernel(x, table):
    x2d = x.astype(jnp.int32).reshape(B // CHUNK, CHUNK)
    out = _lookup(x2d, table)
    return out.reshape(BATCH, HIST, EMBED_DIM)
